# initial kernel scaffold (unmeasured)
import jax
import jax.numpy as jnp
from jax import lax
from jax.experimental import pallas as pl
from jax.experimental.pallas import tpu as pltpu

N_DEV = 8


def _ring_allreduce(x):
    m, n = x.shape
    chunk = m // N_DEV

    def body(x_ref, out_ref, comm_ref, send_sems, recv_sems, step_sem):
        me = lax.axis_index("i")
        left = lax.rem(me + N_DEV - 1, N_DEV)
        right = lax.rem(me + 1, N_DEV)

        barrier = pltpu.get_barrier_semaphore()
        for nbr in (left, right):
            pl.semaphore_signal(
                barrier, inc=1, device_id=(nbr,),
                device_id_type=pl.DeviceIdType.MESH,
            )
        pl.semaphore_wait(barrier, 2)

        out_ref[...] = x_ref[...]

        def step_barrier():
            for nbr in (left, right):
                pl.semaphore_signal(
                    step_sem, inc=1, device_id=(nbr,),
                    device_id_type=pl.DeviceIdType.MESH,
                )
            pl.semaphore_wait(step_sem, 2)

        for s in range(N_DEV - 1):
            slot = s % 2
            send_chunk = lax.rem(me + N_DEV - s, N_DEV)
            recv_chunk = lax.rem(me + 2 * N_DEV - s - 1, N_DEV)
            rdma = pltpu.make_async_remote_copy(
                src_ref=out_ref.at[pl.ds(send_chunk * chunk, chunk), :],
                dst_ref=comm_ref.at[slot],
                send_sem=send_sems.at[slot],
                recv_sem=recv_sems.at[slot],
                device_id=(right,),
                device_id_type=pl.DeviceIdType.MESH,
            )
            rdma.start()
            rdma.wait()
            rows = pl.ds(recv_chunk * chunk, chunk)
            out_ref[rows, :] = (
                out_ref[rows, :].astype(jnp.float32)
                + comm_ref[slot].astype(jnp.float32)
            ).astype(jnp.bfloat16)
            step_barrier()

        for s in range(N_DEV - 1):
            slot = (N_DEV - 1 + s) % 2
            send_chunk = lax.rem(me + 1 + 2 * N_DEV - s, N_DEV)
            rows = pl.ds(send_chunk * chunk, chunk)
            rdma = pltpu.make_async_remote_copy(
                src_ref=out_ref.at[rows, :],
                dst_ref=out_ref.at[rows, :],
                send_sem=send_sems.at[slot],
                recv_sem=recv_sems.at[slot],
                device_id=(right,),
                device_id_type=pl.DeviceIdType.MESH,
            )
            rdma.start()
            rdma.wait()
            step_barrier()

    return pl.pallas_call(
        body,
        out_shape=jax.ShapeDtypeStruct((m, n), jnp.bfloat16),
        in_specs=[pl.BlockSpec(memory_space=pltpu.VMEM)],
        out_specs=pl.BlockSpec(memory_space=pltpu.VMEM),
        scratch_shapes=[
            pltpu.VMEM((2, chunk, n), jnp.bfloat16),
            pltpu.SemaphoreType.DMA((2,)),
            pltpu.SemaphoreType.DMA((2,)),
            pltpu.SemaphoreType.REGULAR,
        ],
        compiler_params=pltpu.CompilerParams(collective_id=0),
    )(x)


def kernel(A, B):
    partial = jnp.dot(
        A.astype(jnp.bfloat16),
        B.astype(jnp.bfloat16),
        preferred_element_type=jnp.float32,
    ).astype(jnp.bfloat16)
    return _ring_allreduce(partial)


# baseline (device time: 833440 ns/iter reference)
import jax
import jax.numpy as jnp
from jax import lax
from jax.experimental import pallas as pl
from jax.experimental.pallas import tpu as pltpu

N_DEV = 8


def _ring_allreduce(x):
    m, n = x.shape
    chunk = m // N_DEV

    def body(x_ref, out_ref, comm_ref, send_sems, recv_sems, step_sem, copy_sem):
        me = lax.axis_index("i")
        left = lax.rem(me + N_DEV - 1, N_DEV)
        right = lax.rem(me + 1, N_DEV)

        copy = pltpu.make_async_copy(x_ref, out_ref, copy_sem)
        copy.start()

        barrier = pltpu.get_barrier_semaphore()
        for nbr in (left, right):
            pl.semaphore_signal(
                barrier, inc=1, device_id=(nbr,),
                device_id_type=pl.DeviceIdType.MESH,
            )
        pl.semaphore_wait(barrier, 2)
        copy.wait()

        def step_barrier():
            for nbr in (left, right):
                pl.semaphore_signal(
                    step_sem, inc=1, device_id=(nbr,),
                    device_id_type=pl.DeviceIdType.MESH,
                )
            pl.semaphore_wait(step_sem, 2)

        for s in range(N_DEV - 1):
            slot = s % 2
            send_chunk = lax.rem(me + N_DEV - s, N_DEV)
            recv_chunk = lax.rem(me + 2 * N_DEV - s - 1, N_DEV)
            rdma = pltpu.make_async_remote_copy(
                src_ref=out_ref.at[pl.ds(send_chunk * chunk, chunk), :],
                dst_ref=comm_ref.at[slot],
                send_sem=send_sems.at[slot],
                recv_sem=recv_sems.at[slot],
                device_id=(right,),
                device_id_type=pl.DeviceIdType.MESH,
            )
            rdma.start()
            rdma.wait()
            rows = pl.ds(recv_chunk * chunk, chunk)
            out_ref[rows, :] = (
                out_ref[rows, :].astype(jnp.float32)
                + comm_ref[slot].astype(jnp.float32)
            ).astype(jnp.bfloat16)
            step_barrier()

        for s in range(N_DEV - 1):
            slot = (N_DEV - 1 + s) % 2
            send_chunk = lax.rem(me + 1 + 2 * N_DEV - s, N_DEV)
            rows = pl.ds(send_chunk * chunk, chunk)
            rdma = pltpu.make_async_remote_copy(
                src_ref=out_ref.at[rows, :],
                dst_ref=out_ref.at[rows, :],
                send_sem=send_sems.at[slot],
                recv_sem=recv_sems.at[slot],
                device_id=(right,),
                device_id_type=pl.DeviceIdType.MESH,
            )
            rdma.start()
            rdma.wait()
            step_barrier()

    return pl.pallas_call(
        body,
        out_shape=jax.ShapeDtypeStruct((m, n), jnp.bfloat16),
        in_specs=[pl.BlockSpec(memory_space=pl.ANY)],
        out_specs=pl.BlockSpec(memory_space=pltpu.VMEM),
        scratch_shapes=[
            pltpu.VMEM((2, chunk, n), jnp.bfloat16),
            pltpu.SemaphoreType.DMA((2,)),
            pltpu.SemaphoreType.DMA((2,)),
            pltpu.SemaphoreType.REGULAR,
            pltpu.SemaphoreType.DMA,
        ],
        compiler_params=pltpu.CompilerParams(
            collective_id=0,
            vmem_limit_bytes=60 * 1024 * 1024,
        ),
    )(x)


def kernel(A, B):
    partial = jnp.dot(
        A.astype(jnp.bfloat16),
        B.astype(jnp.bfloat16),
        preferred_element_type=jnp.float32,
    ).astype(jnp.bfloat16)
    return _ring_allreduce(partial)


# device time: 512606 ns/iter; 1.6259x vs baseline; 1.6259x over previous
import jax
import jax.numpy as jnp
from jax import lax
from jax.experimental import pallas as pl
from jax.experimental.pallas import tpu as pltpu

N_DEV = 8


def _cyc2mesh(c):
    return jnp.where(c < 4, c, 11 - c)


def _ring_allreduce(x):
    m, n = x.shape
    chunk = m // N_DEV
    half = n // 2

    def body(x_ref, out_ref, fcomm, bcomm, fsend, frecv, bsend, brecv,
             step_sem, copy_sem):
        me = lax.axis_index("i")
        cyc = _cyc2mesh(me)
        right = _cyc2mesh(lax.rem(cyc + 1, N_DEV))
        left = _cyc2mesh(lax.rem(cyc + N_DEV - 1, N_DEV))

        copy = pltpu.make_async_copy(x_ref, out_ref, copy_sem)
        copy.start()

        barrier = pltpu.get_barrier_semaphore()
        for nbr in (left, right):
            pl.semaphore_signal(
                barrier, inc=1, device_id=(nbr,),
                device_id_type=pl.DeviceIdType.MESH,
            )
        pl.semaphore_wait(barrier, 2)
        copy.wait()

        def step_barrier():
            for nbr in (left, right):
                pl.semaphore_signal(
                    step_sem, inc=1, device_id=(nbr,),
                    device_id_type=pl.DeviceIdType.MESH,
                )
            pl.semaphore_wait(step_sem, 2)

        def fwd_rows(s):
            c = lax.rem(cyc + 2 * N_DEV - s, N_DEV)
            return pl.ds(c * chunk, chunk)

        def bwd_rows(s):
            c = lax.rem(cyc + 2 * N_DEV + s, N_DEV)
            return pl.ds(c * chunk, chunk)

        lcols = pl.ds(0, half)
        rcols = pl.ds(half, half)

        for s in range(N_DEV - 1):
            slot = s % 2
            f = pltpu.make_async_remote_copy(
                src_ref=out_ref.at[fwd_rows(s), lcols],
                dst_ref=fcomm.at[slot],
                send_sem=fsend.at[slot], recv_sem=frecv.at[slot],
                device_id=(right,), device_id_type=pl.DeviceIdType.MESH,
            )
            b = pltpu.make_async_remote_copy(
                src_ref=out_ref.at[bwd_rows(s), rcols],
                dst_ref=bcomm.at[slot],
                send_sem=bsend.at[slot], recv_sem=brecv.at[slot],
                device_id=(left,), device_id_type=pl.DeviceIdType.MESH,
            )
            f.start()
            b.start()
            f.wait()
            b.wait()
            frows = fwd_rows(s + 1)
            brows = bwd_rows(s + 1)
            out_ref[frows, lcols] = (
                out_ref[frows, lcols].astype(jnp.float32)
                + fcomm[slot].astype(jnp.float32)
            ).astype(jnp.bfloat16)
            out_ref[brows, rcols] = (
                out_ref[brows, rcols].astype(jnp.float32)
                + bcomm[slot].astype(jnp.float32)
            ).astype(jnp.bfloat16)
            step_barrier()

        for s in range(N_DEV - 1):
            slot = (N_DEV - 1 + s) % 2
            frows = fwd_rows(s - 1)
            brows = bwd_rows(s - 1)
            f = pltpu.make_async_remote_copy(
                src_ref=out_ref.at[frows, lcols],
                dst_ref=out_ref.at[frows, lcols],
                send_sem=fsend.at[slot], recv_sem=frecv.at[slot],
                device_id=(right,), device_id_type=pl.DeviceIdType.MESH,
            )
            b = pltpu.make_async_remote_copy(
                src_ref=out_ref.at[brows, rcols],
                dst_ref=out_ref.at[brows, rcols],
                send_sem=bsend.at[slot], recv_sem=brecv.at[slot],
                device_id=(left,), device_id_type=pl.DeviceIdType.MESH,
            )
            f.start()
            b.start()
            f.wait()
            b.wait()
            step_barrier()

    return pl.pallas_call(
        body,
        out_shape=jax.ShapeDtypeStruct((m, n), jnp.bfloat16),
        in_specs=[pl.BlockSpec(memory_space=pl.ANY)],
        out_specs=pl.BlockSpec(memory_space=pltpu.VMEM),
        scratch_shapes=[
            pltpu.VMEM((2, chunk, half), jnp.bfloat16),
            pltpu.VMEM((2, chunk, half), jnp.bfloat16),
            pltpu.SemaphoreType.DMA((2,)),
            pltpu.SemaphoreType.DMA((2,)),
            pltpu.SemaphoreType.DMA((2,)),
            pltpu.SemaphoreType.DMA((2,)),
            pltpu.SemaphoreType.REGULAR,
            pltpu.SemaphoreType.DMA,
        ],
        compiler_params=pltpu.CompilerParams(
            collective_id=0,
            vmem_limit_bytes=60 * 1024 * 1024,
        ),
    )(x)


def kernel(A, B):
    partial = jnp.dot(
        A.astype(jnp.bfloat16),
        B.astype(jnp.bfloat16),
        preferred_element_type=jnp.float32,
    ).astype(jnp.bfloat16)
    return _ring_allreduce(partial)


# device time: 499813 ns/iter; 1.6675x vs baseline; 1.0256x over previous
import jax
import jax.numpy as jnp
from jax import lax
from jax.experimental import pallas as pl
from jax.experimental.pallas import tpu as pltpu

N_DEV = 8


def _cyc2mesh(c):
    return jnp.where(c < 4, c, 11 - c)


def _ring_allreduce(x):
    m, n = x.shape
    chunk = m // N_DEV
    half = n // 2

    def body(x_ref, out_ref, fcomm, bcomm, fsend, frecv, bsend, brecv,
             step_sem, copy_sems):
        me = lax.axis_index("i")
        cyc = _cyc2mesh(me)
        right = _cyc2mesh(lax.rem(cyc + 1, N_DEV))
        left = _cyc2mesh(lax.rem(cyc + N_DEV - 1, N_DEV))

        copies = []
        for j in range(N_DEV):
            rows = pl.ds(lax.rem(cyc + j, N_DEV) * chunk, chunk)
            cp = pltpu.make_async_copy(
                x_ref.at[rows, :], out_ref.at[rows, :], copy_sems.at[j]
            )
            cp.start()
            copies.append(cp)

        barrier = pltpu.get_barrier_semaphore()
        for nbr in (left, right):
            pl.semaphore_signal(
                barrier, inc=1, device_id=(nbr,),
                device_id_type=pl.DeviceIdType.MESH,
            )
        pl.semaphore_wait(barrier, 2)
        copies[0].wait()

        def step_barrier():
            for nbr in (left, right):
                pl.semaphore_signal(
                    step_sem, inc=1, device_id=(nbr,),
                    device_id_type=pl.DeviceIdType.MESH,
                )
            pl.semaphore_wait(step_sem, 2)

        def fwd_rows(s):
            c = lax.rem(cyc + 2 * N_DEV - s, N_DEV)
            return pl.ds(c * chunk, chunk)

        def bwd_rows(s):
            c = lax.rem(cyc + 2 * N_DEV + s, N_DEV)
            return pl.ds(c * chunk, chunk)

        lcols = pl.ds(0, half)
        rcols = pl.ds(half, half)

        for s in range(N_DEV - 1):
            slot = s % 2
            f = pltpu.make_async_remote_copy(
                src_ref=out_ref.at[fwd_rows(s), lcols],
                dst_ref=fcomm.at[slot],
                send_sem=fsend.at[slot], recv_sem=frecv.at[slot],
                device_id=(right,), device_id_type=pl.DeviceIdType.MESH,
            )
            bw = pltpu.make_async_remote_copy(
                src_ref=out_ref.at[bwd_rows(s), rcols],
                dst_ref=bcomm.at[slot],
                send_sem=bsend.at[slot], recv_sem=brecv.at[slot],
                device_id=(left,), device_id_type=pl.DeviceIdType.MESH,
            )
            f.start()
            bw.start()
            if s == 0:
                for cp in copies[1:]:
                    cp.wait()
            frows = fwd_rows(s + 1)
            brows = bwd_rows(s + 1)
            f.wait()
            out_ref[frows, lcols] = (
                out_ref[frows, lcols].astype(jnp.float32)
                + fcomm[slot].astype(jnp.float32)
            ).astype(jnp.bfloat16)
            bw.wait()
            out_ref[brows, rcols] = (
                out_ref[brows, rcols].astype(jnp.float32)
                + bcomm[slot].astype(jnp.float32)
            ).astype(jnp.bfloat16)
            step_barrier()

        for s in range(N_DEV - 1):
            slot = (N_DEV - 1 + s) % 2
            frows = fwd_rows(s - 1)
            brows = bwd_rows(s - 1)
            f = pltpu.make_async_remote_copy(
                src_ref=out_ref.at[frows, lcols],
                dst_ref=out_ref.at[frows, lcols],
                send_sem=fsend.at[slot], recv_sem=frecv.at[slot],
                device_id=(right,), device_id_type=pl.DeviceIdType.MESH,
            )
            bw = pltpu.make_async_remote_copy(
                src_ref=out_ref.at[brows, rcols],
                dst_ref=out_ref.at[brows, rcols],
                send_sem=bsend.at[slot], recv_sem=brecv.at[slot],
                device_id=(left,), device_id_type=pl.DeviceIdType.MESH,
            )
            f.start()
            bw.start()
            f.wait()
            bw.wait()
            step_barrier()

    return pl.pallas_call(
        body,
        out_shape=jax.ShapeDtypeStruct((m, n), jnp.bfloat16),
        in_specs=[pl.BlockSpec(memory_space=pl.ANY)],
        out_specs=pl.BlockSpec(memory_space=pltpu.VMEM),
        scratch_shapes=[
            pltpu.VMEM((2, chunk, half), jnp.bfloat16),
            pltpu.VMEM((2, chunk, half), jnp.bfloat16),
            pltpu.SemaphoreType.DMA((2,)),
            pltpu.SemaphoreType.DMA((2,)),
            pltpu.SemaphoreType.DMA((2,)),
            pltpu.SemaphoreType.DMA((2,)),
            pltpu.SemaphoreType.REGULAR,
            pltpu.SemaphoreType.DMA((N_DEV,)),
        ],
        compiler_params=pltpu.CompilerParams(
            collective_id=0,
            vmem_limit_bytes=60 * 1024 * 1024,
        ),
    )(x)


def kernel(A, B):
    partial = jnp.dot(
        A.astype(jnp.bfloat16),
        B.astype(jnp.bfloat16),
        preferred_element_type=jnp.float32,
    ).astype(jnp.bfloat16)
    return _ring_allreduce(partial)


# device time: 442149 ns/iter; 1.8850x vs baseline; 1.1304x over previous
import jax
import jax.numpy as jnp
from jax import lax
from jax.experimental import pallas as pl
from jax.experimental.pallas import tpu as pltpu

N_DEV = 8


def _cyc2mesh(c):
    return jnp.where(c < 4, c, 11 - c)


def _fused_matmul_allreduce(a, b):
    m, k = a.shape
    _, n = b.shape
    chunk = m // N_DEV
    half = n // 2

    def body(a_ref, b_ref, out_ref, atile, fcomm, bcomm, acopy_sems,
             fsend, frecv, bsend, brecv, step_sem):
        me = lax.axis_index("i")
        cyc = _cyc2mesh(me)
        right = _cyc2mesh(lax.rem(cyc + 1, N_DEV))
        left = _cyc2mesh(lax.rem(cyc + N_DEV - 1, N_DEV))

        lcols = pl.ds(0, half)
        rcols = pl.ds(half, half)

        order = [0, 7, 1, 6, 2, 5, 3, 4]
        chunk_ids = [lax.rem(cyc + o, N_DEV) for o in order]

        copies = [None] * N_DEV

        def start_copy(i):
            rows = pl.ds(chunk_ids[i] * chunk, chunk)
            cp = pltpu.make_async_copy(
                a_ref.at[rows, :], atile.at[i % 2], acopy_sems.at[i % 2]
            )
            cp.start()
            copies[i] = cp

        def compute_chunk(i):
            copies[i].wait()
            if i + 1 < N_DEV:
                start_copy(i + 1)
            rows = pl.ds(chunk_ids[i] * chunk, chunk)
            t = atile[i % 2]
            out_ref[rows, lcols] = jnp.dot(
                t, b_ref[:, lcols], preferred_element_type=jnp.float32
            ).astype(jnp.bfloat16)
            out_ref[rows, rcols] = jnp.dot(
                t, b_ref[:, rcols], preferred_element_type=jnp.float32
            ).astype(jnp.bfloat16)

        start_copy(0)

        barrier = pltpu.get_barrier_semaphore()
        for nbr in (left, right):
            pl.semaphore_signal(
                barrier, inc=1, device_id=(nbr,),
                device_id_type=pl.DeviceIdType.MESH,
            )
        pl.semaphore_wait(barrier, 2)

        compute_chunk(0)

        def step_barrier():
            for nbr in (left, right):
                pl.semaphore_signal(
                    step_sem, inc=1, device_id=(nbr,),
                    device_id_type=pl.DeviceIdType.MESH,
                )
            pl.semaphore_wait(step_sem, 2)

        def fwd_rows(s):
            c = lax.rem(cyc + 2 * N_DEV - s, N_DEV)
            return pl.ds(c * chunk, chunk)

        def bwd_rows(s):
            c = lax.rem(cyc + 2 * N_DEV + s, N_DEV)
            return pl.ds(c * chunk, chunk)

        for s in range(N_DEV - 1):
            slot = s % 2
            f = pltpu.make_async_remote_copy(
                src_ref=out_ref.at[fwd_rows(s), lcols],
                dst_ref=fcomm.at[slot],
                send_sem=fsend.at[slot], recv_sem=frecv.at[slot],
                device_id=(right,), device_id_type=pl.DeviceIdType.MESH,
            )
            bw = pltpu.make_async_remote_copy(
                src_ref=out_ref.at[bwd_rows(s), rcols],
                dst_ref=bcomm.at[slot],
                send_sem=bsend.at[slot], recv_sem=brecv.at[slot],
                device_id=(left,), device_id_type=pl.DeviceIdType.MESH,
            )
            f.start()
            bw.start()
            for i in (2 * s + 1, 2 * s + 2):
                if i < N_DEV:
                    compute_chunk(i)
            frows = fwd_rows(s + 1)
            brows = bwd_rows(s + 1)
            f.wait()
            out_ref[frows, lcols] = (
                out_ref[frows, lcols].astype(jnp.float32)
                + fcomm[slot].astype(jnp.float32)
            ).astype(jnp.bfloat16)
            bw.wait()
            out_ref[brows, rcols] = (
                out_ref[brows, rcols].astype(jnp.float32)
                + bcomm[slot].astype(jnp.float32)
            ).astype(jnp.bfloat16)
            step_barrier()

        for s in range(N_DEV - 1):
            slot = (N_DEV - 1 + s) % 2
            frows = fwd_rows(s - 1)
            brows = bwd_rows(s - 1)
            f = pltpu.make_async_remote_copy(
                src_ref=out_ref.at[frows, lcols],
                dst_ref=out_ref.at[frows, lcols],
                send_sem=fsend.at[slot], recv_sem=frecv.at[slot],
                device_id=(right,), device_id_type=pl.DeviceIdType.MESH,
            )
            bw = pltpu.make_async_remote_copy(
                src_ref=out_ref.at[brows, rcols],
                dst_ref=out_ref.at[brows, rcols],
                send_sem=bsend.at[slot], recv_sem=brecv.at[slot],
                device_id=(left,), device_id_type=pl.DeviceIdType.MESH,
            )
            f.start()
            bw.start()
            f.wait()
            bw.wait()
            step_barrier()

    return pl.pallas_call(
        body,
        out_shape=jax.ShapeDtypeStruct((m, n), jnp.bfloat16),
        in_specs=[
            pl.BlockSpec(memory_space=pl.ANY),
            pl.BlockSpec(memory_space=pltpu.VMEM),
        ],
        out_specs=pl.BlockSpec(memory_space=pltpu.VMEM),
        scratch_shapes=[
            pltpu.VMEM((2, chunk, k), jnp.bfloat16),
            pltpu.VMEM((2, chunk, half), jnp.bfloat16),
            pltpu.VMEM((2, chunk, half), jnp.bfloat16),
            pltpu.SemaphoreType.DMA((2,)),
            pltpu.SemaphoreType.DMA((2,)),
            pltpu.SemaphoreType.DMA((2,)),
            pltpu.SemaphoreType.DMA((2,)),
            pltpu.SemaphoreType.DMA((2,)),
            pltpu.SemaphoreType.REGULAR,
        ],
        compiler_params=pltpu.CompilerParams(
            collective_id=0,
            vmem_limit_bytes=63 * 1024 * 1024,
        ),
    )(a, b)


def kernel(A, B):
    return _fused_matmul_allreduce(
        A.astype(jnp.bfloat16), B.astype(jnp.bfloat16)
    )


# device time: 441870 ns/iter; 1.8862x vs baseline; 1.0006x over previous
import jax
import jax.numpy as jnp
from jax import lax
from jax.experimental import pallas as pl
from jax.experimental.pallas import tpu as pltpu

N_DEV = 8


def _cyc2mesh(c):
    return jnp.where(c < 4, c, 11 - c)


def _fused_matmul_allreduce(a, b):
    m, k = a.shape
    _, n = b.shape
    chunk = m // N_DEV
    half = n // 2

    def body(a_ref, b_ref, out_ref, atile, fcomm, bcomm, acopy_sems,
             fsend, frecv, bsend, brecv, step_sem):
        me = lax.axis_index("i")
        cyc = _cyc2mesh(me)
        right = _cyc2mesh(lax.rem(cyc + 1, N_DEV))
        left = _cyc2mesh(lax.rem(cyc + N_DEV - 1, N_DEV))

        lcols = pl.ds(0, half)
        rcols = pl.ds(half, half)

        def chunk_rows(c):
            return pl.ds(c * chunk, chunk)

        def fwd_chunk(s):
            return lax.rem(cyc + 2 * N_DEV - s, N_DEV)

        def bwd_chunk(s):
            return lax.rem(cyc + 2 * N_DEV + s, N_DEV)

        def tile_copy(c, tslot):
            return pltpu.make_async_copy(
                a_ref.at[chunk_rows(c), :], atile.at[tslot],
                acopy_sems.at[tslot],
            )

        def compute_chunk(c, tslot):
            tile_copy(c, tslot).wait()
            rows = chunk_rows(c)
            t = atile[tslot]
            out_ref[rows, lcols] = jnp.dot(
                t, b_ref[:, lcols], preferred_element_type=jnp.float32
            ).astype(jnp.bfloat16)
            out_ref[rows, rcols] = jnp.dot(
                t, b_ref[:, rcols], preferred_element_type=jnp.float32
            ).astype(jnp.bfloat16)

        tile_copy(cyc, 0).start()

        barrier = pltpu.get_barrier_semaphore()
        for nbr in (left, right):
            pl.semaphore_signal(
                barrier, inc=1, device_id=(nbr,),
                device_id_type=pl.DeviceIdType.MESH,
            )
        pl.semaphore_wait(barrier, 2)

        tile_copy(fwd_chunk(1), 1).start()
        compute_chunk(cyc, 0)

        def step_barrier():
            for nbr in (left, right):
                pl.semaphore_signal(
                    step_sem, inc=1, device_id=(nbr,),
                    device_id_type=pl.DeviceIdType.MESH,
                )
            pl.semaphore_wait(step_sem, 2)

        def rs_step(s, carry):
            slot = lax.rem(s, 2)
            f = pltpu.make_async_remote_copy(
                src_ref=out_ref.at[chunk_rows(fwd_chunk(s)), lcols],
                dst_ref=fcomm.at[slot],
                send_sem=fsend.at[slot], recv_sem=frecv.at[slot],
                device_id=(right,), device_id_type=pl.DeviceIdType.MESH,
            )
            bw = pltpu.make_async_remote_copy(
                src_ref=out_ref.at[chunk_rows(bwd_chunk(s)), rcols],
                dst_ref=bcomm.at[slot],
                send_sem=bsend.at[slot], recv_sem=brecv.at[slot],
                device_id=(left,), device_id_type=pl.DeviceIdType.MESH,
            )
            f.start()
            bw.start()

            @pl.when(s < 4)
            def _():
                @pl.when(s < 3)
                def _():
                    tile_copy(bwd_chunk(s + 1), 0).start()
                compute_chunk(fwd_chunk(s + 1), 1)

            @pl.when(s < 3)
            def _():
                tile_copy(fwd_chunk(s + 2), 1).start()
                compute_chunk(bwd_chunk(s + 1), 0)

            frows = chunk_rows(fwd_chunk(s + 1))
            brows = chunk_rows(bwd_chunk(s + 1))
            f.wait()
            out_ref[frows, lcols] = (
                out_ref[frows, lcols].astype(jnp.float32)
                + fcomm[slot].astype(jnp.float32)
            ).astype(jnp.bfloat16)
            bw.wait()
            out_ref[brows, rcols] = (
                out_ref[brows, rcols].astype(jnp.float32)
                + bcomm[slot].astype(jnp.float32)
            ).astype(jnp.bfloat16)
            step_barrier()
            return carry

        lax.fori_loop(0, N_DEV - 1, rs_step, 0)

        def ag_step(s, carry):
            slot = lax.rem(s + 1, 2)
            frows = chunk_rows(fwd_chunk(s - 1))
            brows = chunk_rows(bwd_chunk(s - 1))
            f = pltpu.make_async_remote_copy(
                src_ref=out_ref.at[frows, lcols],
                dst_ref=out_ref.at[frows, lcols],
                send_sem=fsend.at[slot], recv_sem=frecv.at[slot],
                device_id=(right,), device_id_type=pl.DeviceIdType.MESH,
            )
            bw = pltpu.make_async_remote_copy(
                src_ref=out_ref.at[brows, rcols],
                dst_ref=out_ref.at[brows, rcols],
                send_sem=bsend.at[slot], recv_sem=brecv.at[slot],
                device_id=(left,), device_id_type=pl.DeviceIdType.MESH,
            )
            f.start()
            bw.start()
            f.wait()
            bw.wait()
            step_barrier()
            return carry

        lax.fori_loop(0, N_DEV - 1, ag_step, 0)

    return pl.pallas_call(
        body,
        out_shape=jax.ShapeDtypeStruct((m, n), jnp.bfloat16),
        in_specs=[
            pl.BlockSpec(memory_space=pl.ANY),
            pl.BlockSpec(memory_space=pltpu.VMEM),
        ],
        out_specs=pl.BlockSpec(memory_space=pltpu.VMEM),
        scratch_shapes=[
            pltpu.VMEM((2, chunk, k), jnp.bfloat16),
            pltpu.VMEM((2, chunk, half), jnp.bfloat16),
            pltpu.VMEM((2, chunk, half), jnp.bfloat16),
            pltpu.SemaphoreType.DMA((2,)),
            pltpu.SemaphoreType.DMA((2,)),
            pltpu.SemaphoreType.DMA((2,)),
            pltpu.SemaphoreType.DMA((2,)),
            pltpu.SemaphoreType.DMA((2,)),
            pltpu.SemaphoreType.REGULAR,
        ],
        compiler_params=pltpu.CompilerParams(
            collective_id=0,
            vmem_limit_bytes=67000000,
        ),
    )(a, b)


def kernel(A, B):
    return _fused_matmul_allreduce(
        A.astype(jnp.bfloat16), B.astype(jnp.bfloat16)
    )
